# baseline (speedup 1.0000x reference)
import functools

import jax
import jax.numpy as jnp
from jax import lax
from jax.experimental import pallas as pl
from jax.experimental.pallas import tpu as pltpu
from jax.experimental.pallas import tpu_sc as plsc

N_NODES = 10000
N_PAD = 10112
ROWS_PER_TILE = N_PAD // 16
DUMMY_ROWS = N_PAD - N_NODES
E = 320000
NW = 32
CH = 128
NB = 2
C = 80
E_PAD = NW * CH * C


def _seg_body(F, stage, y_hbm, src_hbm, dst_hbm, out_hbm,
              src_v, dst_v, rows, table, acc, sem0, sem1):
    sems = (sem0, sem1)
    cid = lax.axis_index("c")
    sid = lax.axis_index("s")
    w = sid * 2 + cid

    tcp = None
    if stage:
        trows = N_NODES // 16
        tcp = pltpu.async_copy(y_hbm.at[pl.ds(sid * trows, trows)],
                               table.at[pl.ds(sid * trows, trows)], sem0)
    else:
        table = y_hbm

    def zrow(i, carry):
        for j in range(F // 16):
            rows[0, i, pl.ds(j * 16, 16)] = jnp.zeros((16,), jnp.float32)
        return carry
    lax.fori_loop(0, CH, zrow, 0)
    base = sid * ROWS_PER_TILE
    full, rem = divmod(ROWS_PER_TILE, CH)
    for r in range(full):
        pltpu.sync_copy(rows.at[0], acc.at[pl.ds(base + r * CH, CH)])
    if rem:
        pltpu.sync_copy(rows.at[0, pl.ds(0, rem)],
                        acc.at[pl.ds(base + full * CH, rem)])

    pltpu.sync_copy(src_hbm.at[w], src_v)
    pltpu.sync_copy(dst_hbm.at[w], dst_v)
    if tcp is not None:
        tcp.wait()
    plsc.subcore_barrier()

    for b in range(NB):
        pltpu.async_copy(table.at[src_v.at[b]], rows.at[b], sems[b])

    def outer(g, carry):
        for b in range(NB):
            j = g * NB + b
            pltpu.make_async_copy(table.at[src_v.at[0]], rows.at[b],
                                  sems[b]).wait()
            pltpu.sync_copy(rows.at[b], acc.at[dst_v.at[j]], add=True)
            pltpu.async_copy(table.at[src_v.at[j + NB]], rows.at[b], sems[b])
        return carry
    lax.fori_loop(0, C // NB - 1, outer, 0)
    for b in range(NB):
        j = C - NB + b
        pltpu.make_async_copy(table.at[src_v.at[0]], rows.at[b],
                              sems[b]).wait()
        pltpu.sync_copy(rows.at[b], acc.at[dst_v.at[j]], add=True)

    plsc.subcore_barrier()
    pltpu.sync_copy(acc.at[pl.ds(base, ROWS_PER_TILE)],
                    out_hbm.at[cid, pl.ds(base, ROWS_PER_TILE)])


@functools.lru_cache(maxsize=None)
def _make_segsum(F, stage):
    mesh = plsc.VectorSubcoreMesh(core_axis_name="c", subcore_axis_name="s")
    table_scratch = ([pltpu.VMEM_SHARED((N_NODES, F), jnp.float32)]
                     if stage else [pltpu.VMEM((16,), jnp.float32)])
    return pl.kernel(
        functools.partial(_seg_body, F, stage),
        out_type=jax.ShapeDtypeStruct((2, N_PAD, F), jnp.float32),
        mesh=mesh,
        scratch_types=[
            pltpu.VMEM((C, CH), jnp.int32),
            pltpu.VMEM((C, CH), jnp.int32),
            pltpu.VMEM((NB, CH, F), jnp.float32),
        ] + table_scratch + [
            pltpu.VMEM_SHARED((N_PAD, F), jnp.float32),
            pltpu.SemaphoreType.DMA,
            pltpu.SemaphoreType.DMA,
        ],
        compiler_params=pltpu.CompilerParams(use_tc_tiling_on_sc=False),
        name=f"segsum_f{F}",
    )


def _tc1_body(x_ref, wl_ref, wr_ref, b_ref, y_ref, z_ref):
    x = x_ref[...]
    y_ref[:, :64] = jnp.dot(x, wl_ref[...], preferred_element_type=jnp.float32)
    y_ref[:, 64:] = jnp.ones((N_PAD, 16), jnp.float32)
    z_ref[...] = jnp.dot(x, wr_ref[...], preferred_element_type=jnp.float32) + b_ref[...]


def _tc2_body(p_ref, z_ref, wl_ref, wr_ref, b_ref, inv_ref, y_ref, z2_ref):
    p = p_ref[0] + p_ref[1]
    inv = 1.0 / jnp.maximum(p[:, 64:65], 1.0)
    h = jnp.maximum(p[:, :64] * inv + z_ref[...], 0.0)
    inv_ref[...] = inv
    y_ref[...] = jnp.dot(h, wl_ref[...], preferred_element_type=jnp.float32)
    z2_ref[...] = jnp.dot(h, wr_ref[...], preferred_element_type=jnp.float32) + b_ref[...]


def _tc3_body(p_ref, z_ref, inv_ref, wl_ref, wr_ref, b_ref, y_ref, z3_ref):
    p = p_ref[0] + p_ref[1]
    h = jnp.maximum(p * inv_ref[...] + z_ref[...], 0.0)
    y_ref[...] = jnp.dot(h, wl_ref[...], preferred_element_type=jnp.float32)
    z3_ref[...] = jnp.dot(h, wr_ref[...], preferred_element_type=jnp.float32) + b_ref[...]


def _tc4_body(p_ref, z_ref, inv_ref, wc_ref, bc_ref, out_ref):
    p = p_ref[0] + p_ref[1]
    h = jnp.maximum(p * inv_ref[...] + z_ref[...], 0.0)
    out_ref[...] = jnp.dot(h, wc_ref[...], preferred_element_type=jnp.float32) + bc_ref[...]


_f32 = jnp.float32

_tc1 = pl.pallas_call(
    _tc1_body,
    out_shape=[jax.ShapeDtypeStruct((N_PAD, 80), _f32),
               jax.ShapeDtypeStruct((N_PAD, 64), _f32)])
_tc2 = pl.pallas_call(
    _tc2_body,
    out_shape=[jax.ShapeDtypeStruct((N_PAD, 1), _f32),
               jax.ShapeDtypeStruct((N_PAD, 64), _f32),
               jax.ShapeDtypeStruct((N_PAD, 64), _f32)])
_tc3 = pl.pallas_call(
    _tc3_body,
    out_shape=[jax.ShapeDtypeStruct((N_PAD, 32), _f32),
               jax.ShapeDtypeStruct((N_PAD, 32), _f32)])
_tc4 = pl.pallas_call(
    _tc4_body,
    out_shape=jax.ShapeDtypeStruct((N_PAD, 2), _f32))


def kernel(x, edge_index, Wl1, Wr1, b1, Wl2, Wr2, b2, Wl3, Wr3, b3, Wc, bc):
    src = edge_index[0].astype(jnp.int32)
    dst = edge_index[1].astype(jnp.int32)
    pad = E_PAD - E
    pad_i = jnp.arange(pad, dtype=jnp.int32)
    src3 = jnp.concatenate([src, pad_i % N_NODES]).reshape(NW, C, CH)
    dst3 = jnp.concatenate([dst, N_NODES + pad_i % DUMMY_ROWS]).reshape(NW, C, CH)
    x_p = jnp.pad(x, ((0, N_PAD - N_NODES), (0, 0)))

    y1, z1 = _tc1(x_p, Wl1, Wr1, b1.reshape(1, -1))
    p1 = _make_segsum(80, False)(y1, src3, dst3)
    inv, y2, z2 = _tc2(p1, z1, Wl2, Wr2, b2.reshape(1, -1))
    p2 = _make_segsum(64, True)(y2, src3, dst3)
    y3, z3 = _tc3(p2, z2, inv, Wl3, Wr3, b3.reshape(1, -1))
    p3 = _make_segsum(32, True)(y3, src3, dst3)
    out = _tc4(p3, z3, inv, Wc, bc.reshape(1, -1))
    return out[:N_NODES]


# speedup vs baseline: 1.1518x; 1.1518x over previous
import functools

import jax
import jax.numpy as jnp
from jax import lax
from jax.experimental import pallas as pl
from jax.experimental.pallas import tpu as pltpu
from jax.experimental.pallas import tpu_sc as plsc

N_NODES = 10000
N_PAD = 10112
ROWS_PER_TILE = N_PAD // 16
DUMMY_ROWS = N_PAD - N_NODES
E = 320000
NW = 32
CH = 128
NB = 4
C = 80
E_PAD = NW * CH * C


def _seg_body(F, stage, y_hbm, src_hbm, dst_hbm, out_hbm,
              src_v, dst_v, rows, table, acc, gsems, ssems):
    cid = lax.axis_index("c")
    sid = lax.axis_index("s")
    w = sid * 2 + cid

    tcp = None
    if stage:
        trows = N_NODES // 16
        tcp = pltpu.async_copy(y_hbm.at[pl.ds(sid * trows, trows)],
                               table.at[pl.ds(sid * trows, trows)],
                               gsems.at[0])
    else:
        table = y_hbm

    def zrow(i, carry):
        for j in range(F // 16):
            rows[0, i, pl.ds(j * 16, 16)] = jnp.zeros((16,), jnp.float32)
        return carry
    lax.fori_loop(0, CH, zrow, 0)
    base = sid * ROWS_PER_TILE
    full, rem = divmod(ROWS_PER_TILE, CH)
    for r in range(full):
        pltpu.sync_copy(rows.at[0], acc.at[pl.ds(base + r * CH, CH)])
    if rem:
        pltpu.sync_copy(rows.at[0, pl.ds(0, rem)],
                        acc.at[pl.ds(base + full * CH, rem)])

    pltpu.sync_copy(src_hbm.at[w], src_v)
    pltpu.sync_copy(dst_hbm.at[w], dst_v)
    if tcp is not None:
        tcp.wait()
    plsc.subcore_barrier()

    def wait_gather(b):
        pltpu.make_async_copy(table.at[src_v.at[0]], rows.at[b],
                              gsems.at[b]).wait()

    def wait_scatter(b):
        pltpu.make_async_copy(rows.at[b], acc.at[dst_v.at[0]],
                              ssems.at[b]).wait()

    def issue_gather(j, b):
        pltpu.async_copy(table.at[src_v.at[j]], rows.at[b], gsems.at[b])

    def issue_scatter(j, b):
        pltpu.async_copy(rows.at[b], acc.at[dst_v.at[j]], ssems.at[b],
                         add=True)

    for j in range(NB):
        issue_gather(j, j)

    def outer(g, carry):
        for b in range(NB):
            j = g * NB + b
            wait_gather(b)
            pltpu.sync_copy(rows.at[b], acc.at[dst_v.at[j]], add=True)
            issue_gather(j + NB, b)
        return carry
    lax.fori_loop(0, C // NB - 1, outer, 0)
    for b in range(NB):
        j = C - NB + b
        wait_gather(b)
        pltpu.sync_copy(rows.at[b], acc.at[dst_v.at[j]], add=True)

    plsc.subcore_barrier()
    pltpu.sync_copy(acc.at[pl.ds(base, ROWS_PER_TILE)],
                    out_hbm.at[cid, pl.ds(base, ROWS_PER_TILE)])


@functools.lru_cache(maxsize=None)
def _make_segsum(F, stage):
    mesh = plsc.VectorSubcoreMesh(core_axis_name="c", subcore_axis_name="s")
    table_scratch = ([pltpu.VMEM_SHARED((N_NODES, F), jnp.float32)]
                     if stage else [pltpu.VMEM((16,), jnp.float32)])
    return pl.kernel(
        functools.partial(_seg_body, F, stage),
        out_type=jax.ShapeDtypeStruct((2, N_PAD, F), jnp.float32),
        mesh=mesh,
        scratch_types=[
            pltpu.VMEM((C, CH), jnp.int32),
            pltpu.VMEM((C, CH), jnp.int32),
            pltpu.VMEM((NB, CH, F), jnp.float32),
        ] + table_scratch + [
            pltpu.VMEM_SHARED((N_PAD, F), jnp.float32),
            pltpu.SemaphoreType.DMA((NB,)),
            pltpu.SemaphoreType.DMA((NB,)),
        ],
        compiler_params=pltpu.CompilerParams(use_tc_tiling_on_sc=False),
        name=f"segsum_f{F}",
    )


def _tc1_body(x_ref, wl_ref, wr_ref, b_ref, y_ref, z_ref):
    x = x_ref[...]
    y_ref[:, :64] = jnp.dot(x, wl_ref[...], preferred_element_type=jnp.float32)
    y_ref[:, 64:] = jnp.ones((N_PAD, 16), jnp.float32)
    z_ref[...] = jnp.dot(x, wr_ref[...], preferred_element_type=jnp.float32) + b_ref[...]


def _tc2_body(p_ref, z_ref, wl_ref, wr_ref, b_ref, inv_ref, y_ref, z2_ref):
    p = p_ref[0] + p_ref[1]
    inv = 1.0 / jnp.maximum(p[:, 64:65], 1.0)
    h = jnp.maximum(p[:, :64] * inv + z_ref[...], 0.0)
    inv_ref[...] = inv
    y_ref[...] = jnp.dot(h, wl_ref[...], preferred_element_type=jnp.float32)
    z2_ref[...] = jnp.dot(h, wr_ref[...], preferred_element_type=jnp.float32) + b_ref[...]


def _tc3_body(p_ref, z_ref, inv_ref, wl_ref, wr_ref, b_ref, y_ref, z3_ref):
    p = p_ref[0] + p_ref[1]
    h = jnp.maximum(p * inv_ref[...] + z_ref[...], 0.0)
    y_ref[...] = jnp.dot(h, wl_ref[...], preferred_element_type=jnp.float32)
    z3_ref[...] = jnp.dot(h, wr_ref[...], preferred_element_type=jnp.float32) + b_ref[...]


def _tc4_body(p_ref, z_ref, inv_ref, wc_ref, bc_ref, out_ref):
    p = p_ref[0] + p_ref[1]
    h = jnp.maximum(p * inv_ref[...] + z_ref[...], 0.0)
    out_ref[...] = jnp.dot(h, wc_ref[...], preferred_element_type=jnp.float32) + bc_ref[...]


_f32 = jnp.float32

_tc1 = pl.pallas_call(
    _tc1_body,
    out_shape=[jax.ShapeDtypeStruct((N_PAD, 80), _f32),
               jax.ShapeDtypeStruct((N_PAD, 64), _f32)])
_tc2 = pl.pallas_call(
    _tc2_body,
    out_shape=[jax.ShapeDtypeStruct((N_PAD, 1), _f32),
               jax.ShapeDtypeStruct((N_PAD, 64), _f32),
               jax.ShapeDtypeStruct((N_PAD, 64), _f32)])
_tc3 = pl.pallas_call(
    _tc3_body,
    out_shape=[jax.ShapeDtypeStruct((N_PAD, 32), _f32),
               jax.ShapeDtypeStruct((N_PAD, 32), _f32)])
_tc4 = pl.pallas_call(
    _tc4_body,
    out_shape=jax.ShapeDtypeStruct((N_PAD, 2), _f32))


def kernel(x, edge_index, Wl1, Wr1, b1, Wl2, Wr2, b2, Wl3, Wr3, b3, Wc, bc):
    src = edge_index[0].astype(jnp.int32)
    dst = edge_index[1].astype(jnp.int32)
    pad = E_PAD - E
    pad_i = jnp.arange(pad, dtype=jnp.int32)
    src3 = jnp.concatenate([src, pad_i % N_NODES]).reshape(NW, C, CH)
    dst3 = jnp.concatenate([dst, N_NODES + pad_i % DUMMY_ROWS]).reshape(NW, C, CH)
    x_p = jnp.pad(x, ((0, N_PAD - N_NODES), (0, 0)))

    y1, z1 = _tc1(x_p, Wl1, Wr1, b1.reshape(1, -1))
    p1 = _make_segsum(80, False)(y1, src3, dst3)
    inv, y2, z2 = _tc2(p1, z1, Wl2, Wr2, b2.reshape(1, -1))
    p2 = _make_segsum(64, False)(y2, src3, dst3)
    y3, z3 = _tc3(p2, z2, inv, Wl3, Wr3, b3.reshape(1, -1))
    p3 = _make_segsum(32, True)(y3, src3, dst3)
    out = _tc4(p3, z3, inv, Wc, bc.reshape(1, -1))
    return out[:N_NODES]


# speedup vs baseline: 1.2022x; 1.0438x over previous
import functools

import jax
import jax.numpy as jnp
import numpy as np
from jax import lax
from jax.experimental import pallas as pl
from jax.experimental.pallas import tpu as pltpu
from jax.experimental.pallas import tpu_sc as plsc

N_NODES = 10000
N_PAD = 10112
ROWS_PER_TILE = N_PAD // 16
DUMMY_ROWS = N_PAD - N_NODES
E = 320000
NW = 32
CH = 128
NB = 4
C = 80
E_PAD = NW * CH * C


def _seg_body(F, stage, y_hbm, src_hbm, dst_hbm, out_hbm,
              src_v, dst_v, rows, table, acc, gsems, ssems):
    cid = lax.axis_index("c")
    sid = lax.axis_index("s")
    w = sid * 2 + cid

    tcp = None
    if stage:
        trows = N_NODES // 16
        tcp = pltpu.async_copy(y_hbm.at[pl.ds(sid * trows, trows)],
                               table.at[pl.ds(sid * trows, trows)],
                               gsems.at[0])
    else:
        table = y_hbm

    def zrow(i, carry):
        for j in range(F // 16):
            rows[0, i, pl.ds(j * 16, 16)] = jnp.zeros((16,), jnp.float32)
        return carry
    lax.fori_loop(0, CH, zrow, 0)
    base = sid * ROWS_PER_TILE
    full, rem = divmod(ROWS_PER_TILE, CH)
    for r in range(full):
        pltpu.sync_copy(rows.at[0], acc.at[pl.ds(base + r * CH, CH)])
    if rem:
        pltpu.sync_copy(rows.at[0, pl.ds(0, rem)],
                        acc.at[pl.ds(base + full * CH, rem)])

    pltpu.sync_copy(src_hbm.at[w], src_v)
    pltpu.sync_copy(dst_hbm.at[w], dst_v)
    if tcp is not None:
        tcp.wait()
    plsc.subcore_barrier()

    def wait_gather(b):
        pltpu.make_async_copy(table.at[src_v.at[0]], rows.at[b],
                              gsems.at[b]).wait()

    def wait_scatter(b):
        pltpu.make_async_copy(rows.at[b], acc.at[dst_v.at[0]],
                              ssems.at[b]).wait()

    def issue_gather(j, b):
        pltpu.async_copy(table.at[src_v.at[j]], rows.at[b], gsems.at[b])

    def issue_scatter(j, b):
        pltpu.async_copy(rows.at[b], acc.at[dst_v.at[j]], ssems.at[b],
                         add=True)

    for j in range(NB):
        issue_gather(j, j)

    def outer(g, carry):
        for b in range(NB):
            j = g * NB + b
            wait_gather(b)
            pltpu.sync_copy(rows.at[b], acc.at[dst_v.at[j]], add=True)
            issue_gather(j + NB, b)
        return carry
    lax.fori_loop(0, C // NB - 1, outer, 0)
    for b in range(NB):
        j = C - NB + b
        wait_gather(b)
        pltpu.sync_copy(rows.at[b], acc.at[dst_v.at[j]], add=True)

    plsc.subcore_barrier()
    pltpu.sync_copy(acc.at[pl.ds(base, ROWS_PER_TILE)],
                    out_hbm.at[cid, pl.ds(base, ROWS_PER_TILE)])


@functools.lru_cache(maxsize=None)
def _make_segsum(F, stage):
    mesh = plsc.VectorSubcoreMesh(core_axis_name="c", subcore_axis_name="s")
    table_scratch = ([pltpu.VMEM_SHARED((N_NODES, F), jnp.float32)]
                     if stage else [pltpu.VMEM((16,), jnp.float32)])
    return pl.kernel(
        functools.partial(_seg_body, F, stage),
        out_type=jax.ShapeDtypeStruct((2, N_PAD, F), jnp.float32),
        mesh=mesh,
        scratch_types=[
            pltpu.VMEM((C, CH), jnp.int32),
            pltpu.VMEM((C, CH), jnp.int32),
            pltpu.VMEM((NB, CH, F), jnp.float32),
        ] + table_scratch + [
            pltpu.VMEM_SHARED((N_PAD, F), jnp.float32),
            pltpu.SemaphoreType.DMA((NB,)),
            pltpu.SemaphoreType.DMA((NB,)),
        ],
        compiler_params=pltpu.CompilerParams(use_tc_tiling_on_sc=False),
        name=f"segsum_f{F}",
    )


def _tc1_body(x_ref, wl_ref, wr_ref, b_ref, y_ref, z_ref):
    x = x_ref[...]
    y_ref[:, :64] = jnp.dot(x, wl_ref[...], preferred_element_type=jnp.float32)
    y_ref[:, 64:] = jnp.ones((N_NODES, 16), jnp.float32)
    z_ref[...] = jnp.dot(x, wr_ref[...], preferred_element_type=jnp.float32) + b_ref[...]


def _tc2_body(p_ref, z_ref, wl_ref, wr_ref, b_ref, inv_ref, y_ref, z2_ref):
    p = p_ref[0, :N_NODES, :] + p_ref[1, :N_NODES, :]
    inv = 1.0 / jnp.maximum(p[:, 64:65], 1.0)
    h = jnp.maximum(p[:, :64] * inv + z_ref[...], 0.0)
    inv_ref[...] = inv
    y_ref[...] = jnp.dot(h, wl_ref[...], preferred_element_type=jnp.float32)
    z2_ref[...] = jnp.dot(h, wr_ref[...], preferred_element_type=jnp.float32) + b_ref[...]


def _tc3_body(p_ref, z_ref, inv_ref, wl_ref, wr_ref, b_ref, y_ref, z3_ref):
    p = p_ref[0, :N_NODES, :] + p_ref[1, :N_NODES, :]
    h = jnp.maximum(p * inv_ref[...] + z_ref[...], 0.0)
    y_ref[...] = jnp.dot(h, wl_ref[...], preferred_element_type=jnp.float32)
    z3_ref[...] = jnp.dot(h, wr_ref[...], preferred_element_type=jnp.float32) + b_ref[...]


def _tc4_body(p_ref, z_ref, inv_ref, wc_ref, bc_ref, out_ref):
    p = p_ref[0, :N_NODES, :] + p_ref[1, :N_NODES, :]
    h = jnp.maximum(p * inv_ref[...] + z_ref[...], 0.0)
    out_ref[...] = jnp.dot(h, wc_ref[...], preferred_element_type=jnp.float32) + bc_ref[...]


_f32 = jnp.float32

_tc1 = pl.pallas_call(
    _tc1_body,
    out_shape=[jax.ShapeDtypeStruct((N_NODES, 80), _f32),
               jax.ShapeDtypeStruct((N_NODES, 64), _f32)])
_tc2 = pl.pallas_call(
    _tc2_body,
    out_shape=[jax.ShapeDtypeStruct((N_NODES, 1), _f32),
               jax.ShapeDtypeStruct((N_NODES, 64), _f32),
               jax.ShapeDtypeStruct((N_NODES, 64), _f32)])
_tc3 = pl.pallas_call(
    _tc3_body,
    out_shape=[jax.ShapeDtypeStruct((N_NODES, 32), _f32),
               jax.ShapeDtypeStruct((N_NODES, 32), _f32)])
_tc4 = pl.pallas_call(
    _tc4_body,
    out_shape=jax.ShapeDtypeStruct((N_NODES, 2), _f32))


_PAD_SRC = jnp.asarray(np.arange(E_PAD - E, dtype=np.int32) % N_NODES)
_PAD_DST = jnp.asarray(N_NODES + np.arange(E_PAD - E, dtype=np.int32) % DUMMY_ROWS)


def kernel(x, edge_index, Wl1, Wr1, b1, Wl2, Wr2, b2, Wl3, Wr3, b3, Wc, bc):
    src = edge_index[0].astype(jnp.int32)
    dst = edge_index[1].astype(jnp.int32)
    src3 = jnp.concatenate([src, _PAD_SRC]).reshape(NW, C, CH)
    dst3 = jnp.concatenate([dst, _PAD_DST]).reshape(NW, C, CH)

    y1, z1 = _tc1(x, Wl1, Wr1, b1.reshape(1, -1))
    p1 = _make_segsum(80, False)(y1, src3, dst3)
    inv, y2, z2 = _tc2(p1, z1, Wl2, Wr2, b2.reshape(1, -1))
    p2 = _make_segsum(64, False)(y2, src3, dst3)
    y3, z3 = _tc3(p2, z2, inv, Wl3, Wr3, b3.reshape(1, -1))
    p3 = _make_segsum(32, False)(y3, src3, dst3)
    return _tc4(p3, z3, inv, Wc, bc.reshape(1, -1))


# speedup vs baseline: 1.3293x; 1.1057x over previous
import functools

import jax
import jax.numpy as jnp
import numpy as np
from jax import lax
from jax.experimental import pallas as pl
from jax.experimental.pallas import tpu as pltpu
from jax.experimental.pallas import tpu_sc as plsc

N_NODES = 10000
N_PAD = 10112
ROWS_PER_TILE = N_PAD // 16
DUMMY_ROWS = N_PAD - N_NODES
E = 320000
NW = 32
CH = 128
NB = 4
C = 80
E_PAD = NW * CH * C


def _seg_body(F, stage, y_hbm, src_hbm, dst_hbm, out_hbm,
              src_v, dst_v, rows, table, acc, gsems, ssems):
    cid = lax.axis_index("c")
    sid = lax.axis_index("s")
    w = sid * 2 + cid

    tcp = None
    if stage:
        trows = N_NODES // 16
        tcp = pltpu.async_copy(y_hbm.at[pl.ds(sid * trows, trows)],
                               table.at[pl.ds(sid * trows, trows)],
                               gsems.at[0])
    else:
        table = y_hbm

    def zrow(i, carry):
        for j in range(F // 16):
            rows[0, i, pl.ds(j * 16, 16)] = jnp.zeros((16,), jnp.float32)
        return carry
    lax.fori_loop(0, CH, zrow, 0)
    base = sid * ROWS_PER_TILE
    full, rem = divmod(ROWS_PER_TILE, CH)
    for r in range(full):
        pltpu.sync_copy(rows.at[0], acc.at[pl.ds(base + r * CH, CH)])
    if rem:
        pltpu.sync_copy(rows.at[0, pl.ds(0, rem)],
                        acc.at[pl.ds(base + full * CH, rem)])

    pltpu.sync_copy(src_hbm.at[w], src_v)
    pltpu.sync_copy(dst_hbm.at[w], dst_v)
    if tcp is not None:
        tcp.wait()
    plsc.subcore_barrier()

    def wait_gather(b):
        pltpu.make_async_copy(table.at[src_v.at[0]], rows.at[b],
                              gsems.at[b]).wait()

    def issue_gather(j, b):
        pltpu.async_copy(table.at[src_v.at[j]], rows.at[b], gsems.at[b])

    for j in range(NB):
        issue_gather(j, j)

    def outer(g, carry):
        for b in range(NB):
            j = g * NB + b
            wait_gather(b)
            pltpu.sync_copy(rows.at[b], acc.at[dst_v.at[j]], add=True)
            issue_gather(j + NB, b)
        return carry
    lax.fori_loop(0, C // NB - 1, outer, 0)
    for b in range(NB):
        j = C - NB + b
        wait_gather(b)
        pltpu.sync_copy(rows.at[b], acc.at[dst_v.at[j]], add=True)

    plsc.subcore_barrier()
    pltpu.sync_copy(acc.at[pl.ds(base, ROWS_PER_TILE)],
                    out_hbm.at[cid, pl.ds(base, ROWS_PER_TILE), pl.ds(0, F)])


@functools.lru_cache(maxsize=None)
def _make_segsum(F, stage):
    mesh = plsc.VectorSubcoreMesh(core_axis_name="c", subcore_axis_name="s")
    table_scratch = ([pltpu.VMEM_SHARED((N_NODES, F), jnp.float32)]
                     if stage else [pltpu.VMEM((16,), jnp.float32)])
    return pl.kernel(
        functools.partial(_seg_body, F, stage),
        out_type=jax.ShapeDtypeStruct((2, N_PAD, 128), jnp.float32),
        mesh=mesh,
        scratch_types=[
            pltpu.VMEM((C, CH), jnp.int32),
            pltpu.VMEM((C, CH), jnp.int32),
            pltpu.VMEM((NB, CH, F), jnp.float32),
        ] + table_scratch + [
            pltpu.VMEM_SHARED((N_PAD, F), jnp.float32),
            pltpu.SemaphoreType.DMA((NB,)),
            pltpu.SemaphoreType.DMA((NB,)),
        ],
        compiler_params=pltpu.CompilerParams(use_tc_tiling_on_sc=False),
        name=f"segsum_f{F}",
    )


def _tc1_body(x_ref, wl_ref, wr_ref, b_ref, y_ref, z_ref):
    x = x_ref[...]
    y_ref[:, :64] = jnp.dot(x, wl_ref[...], preferred_element_type=jnp.float32)
    y_ref[:, 64:] = jnp.ones((N_NODES, 16), jnp.float32)
    z_ref[...] = jnp.dot(x, wr_ref[...], preferred_element_type=jnp.float32) + b_ref[...]


def _tc2_body(p_ref, z_ref, wl_ref, wr_ref, b_ref, inv_ref, y_ref, z2_ref):
    p = p_ref[0, :N_NODES, :] + p_ref[1, :N_NODES, :]
    inv = 1.0 / jnp.maximum(p[:, 64:65], 1.0)
    h = jnp.maximum(p[:, :64] * inv + z_ref[...], 0.0)
    inv_ref[...] = inv
    y_ref[...] = jnp.dot(h, wl_ref[...], preferred_element_type=jnp.float32)
    z2_ref[...] = jnp.dot(h, wr_ref[...], preferred_element_type=jnp.float32) + b_ref[...]


def _tc3_body(p_ref, z_ref, inv_ref, wl_ref, wr_ref, b_ref, y_ref, z3_ref):
    p = p_ref[0, :N_NODES, :64] + p_ref[1, :N_NODES, :64]
    h = jnp.maximum(p * inv_ref[...] + z_ref[...], 0.0)
    y_ref[...] = jnp.dot(h, wl_ref[...], preferred_element_type=jnp.float32)
    z3_ref[...] = jnp.dot(h, wr_ref[...], preferred_element_type=jnp.float32) + b_ref[...]


def _tc4_body(p_ref, z_ref, inv_ref, wc_ref, bc_ref, out_ref):
    p = p_ref[0, :N_NODES, :32] + p_ref[1, :N_NODES, :32]
    h = jnp.maximum(p * inv_ref[...] + z_ref[...], 0.0)
    out_ref[...] = jnp.dot(h, wc_ref[...], preferred_element_type=jnp.float32) + bc_ref[...]


_f32 = jnp.float32

_tc1 = pl.pallas_call(
    _tc1_body,
    out_shape=[jax.ShapeDtypeStruct((N_NODES, 80), _f32),
               jax.ShapeDtypeStruct((N_NODES, 64), _f32)])
_tc2 = pl.pallas_call(
    _tc2_body,
    out_shape=[jax.ShapeDtypeStruct((N_NODES, 1), _f32),
               jax.ShapeDtypeStruct((N_NODES, 64), _f32),
               jax.ShapeDtypeStruct((N_NODES, 64), _f32)])
_tc3 = pl.pallas_call(
    _tc3_body,
    out_shape=[jax.ShapeDtypeStruct((N_NODES, 32), _f32),
               jax.ShapeDtypeStruct((N_NODES, 32), _f32)])
_tc4 = pl.pallas_call(
    _tc4_body,
    out_shape=jax.ShapeDtypeStruct((N_NODES, 2), _f32))


_PAD_SRC = jnp.asarray(np.arange(E_PAD - E, dtype=np.int32) % N_NODES)
_PAD_DST = jnp.asarray(N_NODES + np.arange(E_PAD - E, dtype=np.int32) % DUMMY_ROWS)


def kernel(x, edge_index, Wl1, Wr1, b1, Wl2, Wr2, b2, Wl3, Wr3, b3, Wc, bc):
    src = edge_index[0].astype(jnp.int32)
    dst = edge_index[1].astype(jnp.int32)
    src3 = jnp.concatenate([src, _PAD_SRC]).reshape(NW, C, CH)
    dst3 = jnp.concatenate([dst, _PAD_DST]).reshape(NW, C, CH)

    y1, z1 = _tc1(x, Wl1, Wr1, b1.reshape(1, -1))
    p1 = _make_segsum(80, False)(y1, src3, dst3)
    inv, y2, z2 = _tc2(p1, z1, Wl2, Wr2, b2.reshape(1, -1))
    p2 = _make_segsum(64, False)(y2, src3, dst3)
    y3, z3 = _tc3(p2, z2, inv, Wl3, Wr3, b3.reshape(1, -1))
    p3 = _make_segsum(32, False)(y3, src3, dst3)
    return _tc4(p3, z3, inv, Wc, bc.reshape(1, -1))
